# baseline (speedup 1.0000x reference)
import jax
import jax.numpy as jnp
from jax import lax
from jax.experimental import pallas as pl
from jax.experimental.pallas import tpu as pltpu

N_USERS = 4096
NUM_SUBREDDITS = 20000
X_DIM = 64
SUB_REP_DIM = 3

BK = 256
NSTEPS = 78
KMAIN = NSTEPS * BK
KREM = NUM_SUBREDDITS - KMAIN


def _agg_kernel(st_ref, strem_ref, rt_ref, o_ref):
    i = pl.program_id(0)
    st = st_ref[...].astype(jnp.bfloat16)
    rt = rt_ref[:, pl.ds(i * BK, BK)].astype(jnp.bfloat16)
    acc = lax.dot_general(
        rt, st,
        dimension_numbers=(((1,), (0,)), ((), ())),
        preferred_element_type=jnp.float32,
    )

    @pl.when(i == 0)
    def _():
        rem = lax.dot_general(
            rt_ref[:, KMAIN:].astype(jnp.bfloat16),
            strem_ref[...].astype(jnp.bfloat16),
            dimension_numbers=(((1,), (0,)), ((), ())),
            preferred_element_type=jnp.float32,
        )
        o_ref[...] = acc + rem

    @pl.when(i != 0)
    def _():
        o_ref[...] = o_ref[...] + acc


def kernel(x, S, R):
    ST = S.T
    RT = R.T
    o_t = pl.pallas_call(
        _agg_kernel,
        grid=(NSTEPS,),
        in_specs=[
            pl.BlockSpec((BK, N_USERS), lambda i: (i, 0)),
            pl.BlockSpec((KREM, N_USERS), lambda i: (NSTEPS * (BK // KREM), 0)),
            pl.BlockSpec((SUB_REP_DIM, NUM_SUBREDDITS), lambda i: (0, 0)),
        ],
        out_specs=pl.BlockSpec((SUB_REP_DIM, N_USERS), lambda i: (0, 0)),
        out_shape=jax.ShapeDtypeStruct((SUB_REP_DIM, N_USERS), jnp.float32),
        compiler_params=pltpu.CompilerParams(
            dimension_semantics=("arbitrary",),
            vmem_limit_bytes=100 * 1024 * 1024,
        ),
    )(ST, ST, RT)
    return jnp.concatenate((x, o_t.T), axis=1)


# speedup vs baseline: 1.1314x; 1.1314x over previous
import jax
import jax.numpy as jnp
from jax import lax
from jax.experimental import pallas as pl
from jax.experimental.pallas import tpu as pltpu

N_USERS = 4096
NUM_SUBREDDITS = 20000
X_DIM = 64
SUB_REP_DIM = 3

BK = 768
NSTEPS = 26
KMAIN = NSTEPS * BK
KREM = NUM_SUBREDDITS - KMAIN


def _agg_kernel(st_ref, strem_ref, rt_ref, o_ref):
    i = pl.program_id(0)
    st = st_ref[...].astype(jnp.bfloat16)
    rt = rt_ref[:, pl.ds(i * BK, BK)].astype(jnp.bfloat16)
    acc = lax.dot_general(
        rt, st,
        dimension_numbers=(((1,), (0,)), ((), ())),
        preferred_element_type=jnp.float32,
    )

    @pl.when(i == 0)
    def _():
        rem = lax.dot_general(
            rt_ref[:, KMAIN:].astype(jnp.bfloat16),
            strem_ref[...].astype(jnp.bfloat16),
            dimension_numbers=(((1,), (0,)), ((), ())),
            preferred_element_type=jnp.float32,
        )
        o_ref[...] = acc + rem

    @pl.when(i != 0)
    def _():
        o_ref[...] = o_ref[...] + acc


def kernel(x, S, R):
    ST = S.T
    RT = R.T
    o_t = pl.pallas_call(
        _agg_kernel,
        grid=(NSTEPS,),
        in_specs=[
            pl.BlockSpec((BK, N_USERS), lambda i: (i, 0)),
            pl.BlockSpec((KREM, N_USERS), lambda i: (NSTEPS * (BK // KREM), 0)),
            pl.BlockSpec((SUB_REP_DIM, NUM_SUBREDDITS), lambda i: (0, 0)),
        ],
        out_specs=pl.BlockSpec((SUB_REP_DIM, N_USERS), lambda i: (0, 0)),
        out_shape=jax.ShapeDtypeStruct((SUB_REP_DIM, N_USERS), jnp.float32),
        compiler_params=pltpu.CompilerParams(
            dimension_semantics=("arbitrary",),
            vmem_limit_bytes=100 * 1024 * 1024,
        ),
    )(ST, ST, RT)
    return jnp.concatenate((x, o_t.T), axis=1)
